# baseline (speedup 1.0000x reference)
import functools

import jax
import jax.numpy as jnp
from jax import lax
from jax.experimental import pallas as pl
from jax.experimental.pallas import tpu as pltpu
from jax.experimental.pallas import tpu_sc as plsc

P = 100000
NN = 8
T = 16
L = 16
C = 80
NW = 32

H = 0.01
EPS = 1e-8
INV_H_HALF = 0.5 / H
INV_H2 = 1.0 / (H * H)


def _make_sc_kernel(num_points, interpret=False):
  groups = C // L
  nchunks = num_points // C
  base_chunks = nchunks // NW
  extra = nchunks % NW
  dlen = C * 3 * NN

  def _sc_body(uT, idx2, d1f, d2f, Dh, ch, outT,
               idx_v, g_v, dD_v, d1_v, d2_v, c_v, Dow_v, uo_v, out_v,
               isem0, isem1, dsem0, dsem1, gsem0, gsem1, osem0, osem1):
    isem = (isem0, isem1)
    dsem = (dsem0, dsem1)
    gsem = (gsem0, gsem1)
    osem = (osem0, osem1)
    wid = lax.axis_index("s") * 2 + lax.axis_index("c")
    nch = base_chunks + (wid < extra).astype(jnp.int32)
    iota = lax.iota(jnp.int32, L)

    def issue_inputs(chunk, b):
      base = chunk * C
      pltpu.async_copy(
          idx2.at[pl.ds(chunk * groups, groups)], idx_v.at[b], isem[b])
      pltpu.async_copy(d1f.at[pl.ds(base * 3 * NN, dlen)], d1_v.at[b],
                       dsem[b])
      pltpu.async_copy(d2f.at[pl.ds(base * 3 * NN, dlen)], d2_v.at[b],
                       dsem[b])
      pltpu.async_copy(ch.at[pl.ds(base, C)], c_v.at[b], dsem[b])
      pltpu.async_copy(Dh.at[pl.ds(base, C)], Dow_v.at[b], dsem[b])
      pltpu.async_copy(uT.at[pl.ds(base, C)], uo_v.at[b], dsem[b])

    def wait_idx(b):
      pltpu.make_async_copy(
          idx2.at[pl.ds(0, groups)], idx_v.at[b], isem[b]).wait()

    def issue_gather(b):
      for g in range(groups):
        pltpu.async_copy(uT.at[idx_v.at[b, g]], g_v.at[b, g], gsem[b])
        pltpu.async_copy(Dh.at[idx_v.at[b, g]], dD_v.at[b, g], gsem[b])

    def wait_gather(b):
      for g in range(groups):
        pltpu.make_async_copy(
            uT.at[pl.ds(0, 128)], g_v.at[b, g], gsem[b]).wait()
        pltpu.make_async_copy(
            Dh.at[pl.ds(0, 128)], dD_v.at[b, g], gsem[b]).wait()

    def wait_inputs(b):
      pltpu.make_async_copy(d1f.at[pl.ds(0, dlen)], d1_v.at[b],
                            dsem[b]).wait()
      pltpu.make_async_copy(d2f.at[pl.ds(0, dlen)], d2_v.at[b],
                            dsem[b]).wait()
      pltpu.make_async_copy(ch.at[pl.ds(0, C)], c_v.at[b], dsem[b]).wait()
      pltpu.make_async_copy(Dh.at[pl.ds(0, C)], Dow_v.at[b], dsem[b]).wait()
      pltpu.make_async_copy(uT.at[pl.ds(0, C)], uo_v.at[b], dsem[b]).wait()

    def issue_out(chunk, b):
      pltpu.async_copy(out_v.at[b], outT.at[pl.ds(chunk * C, C)], osem[b])

    def wait_out(b):
      pltpu.make_async_copy(
          out_v.at[b], outT.at[pl.ds(0, C)], osem[b]).wait()

    def compute(chunk, b):
      base = chunk * C
      for g in range(groups):
        plane = g * L + iota

        def axis_weights(dref):
          ws = []
          for i in range(3):
            r = [1.0 / (plsc.load_gather(
                    dref, [plane * (3 * NN) + (i * NN + n)]) + EPS)
                 for n in range(NN)]
            s = r[0]
            for n in range(1, NN):
              s = s + r[n]
            inv = 1.0 / s
            ws.append([x * inv for x in r])
          a = [(ws[2][n] - ws[0][n]) * INV_H_HALF for n in range(NN)]
          b_ = [(ws[0][n] - 2.0 * ws[1][n] + ws[2][n]) * INV_H2
                for n in range(NN)]
          return a, b_

        a1, b1 = axis_weights(d1_v.at[b])
        a2, b2 = axis_weights(d2_v.at[b])

        j8 = [iota * NN + n for n in range(NN)]
        Dg = [plsc.load_gather(dD_v.at[b, g], [j8[n]]) for n in range(NN)]
        dD1 = a1[0] * Dg[0]
        dD2 = a2[0] * Dg[0]
        for n in range(1, NN):
          dD1 = dD1 + a1[n] * Dg[n]
          dD2 = dD2 + a2[n] * Dg[n]
        Down = plsc.load_gather(Dow_v.at[b], [plane])
        W = [dD1 * a1[n] + dD2 * a2[n] + Down * (b1[n] + b2[n])
             for n in range(NN)]
        cown = plsc.load_gather(c_v.at[b], [plane])

        for tt in range(T):
          tfull = jnp.full((L,), tt, jnp.int32)
          acc = cown * plsc.load_gather(uo_v.at[b], [plane, tfull])
          for n in range(NN):
            gv = plsc.load_gather(g_v.at[b, g], [j8[n], tfull])
            acc = acc + W[n] * gv
          plsc.store_scatter(out_v.at[b], [plane, tfull], acc)

    issue_inputs(wid, 0)
    wait_idx(0)
    issue_gather(0)

    niter = (nch + 1) // 2

    def loop_body(k, carry):
      c0 = wid + NW * (2 * k)
      c1 = wid + NW * (2 * k + 1)
      c2 = wid + NW * (2 * k + 2)
      g1 = (2 * k + 1) < nch
      g2 = (2 * k + 2) < nch

      pl.when(g1)(lambda: issue_inputs(c1, 1))
      wait_inputs(0)
      wait_gather(0)

      def _start_g1():
        wait_idx(1)
        issue_gather(1)
      pl.when(g1)(_start_g1)

      pl.when(k > 0)(lambda: wait_out(0))
      compute(c0, 0)
      issue_out(c0, 0)

      def _half1():
        pl.when(g2)(lambda: issue_inputs(c2, 0))
        wait_inputs(1)
        wait_gather(1)
        pl.when(k > 0)(lambda: wait_out(1))
        compute(c1, 1)
        issue_out(c1, 1)

        def _start_g2():
          wait_idx(0)
          issue_gather(0)
        pl.when(g2)(_start_g2)
      pl.when(g1)(_half1)
      return carry

    lax.fori_loop(0, niter, loop_body, jnp.int32(0))
    wait_out(0)
    wait_out(1)

  return functools.partial(
      pl.kernel,
      out_type=jax.ShapeDtypeStruct((num_points, T), jnp.float32),
      mesh=plsc.VectorSubcoreMesh(core_axis_name="c", subcore_axis_name="s",
                                  num_cores=2, num_subcores=16),
      compiler_params=pltpu.CompilerParams(
          needs_layout_passes=False, use_tc_tiling_on_sc=False),
      interpret=interpret,
      scratch_types=[
          pltpu.VMEM((2, groups, 128), jnp.int32),
          pltpu.VMEM((2, groups, 128, T), jnp.float32),
          pltpu.VMEM((2, groups, 128), jnp.float32),
          pltpu.VMEM((2, dlen), jnp.float32),
          pltpu.VMEM((2, dlen), jnp.float32),
          pltpu.VMEM((2, C), jnp.float32),
          pltpu.VMEM((2, C), jnp.float32),
          pltpu.VMEM((2, C, T), jnp.float32),
          pltpu.VMEM((2, C, T), jnp.float32),
          pltpu.SemaphoreType.DMA,
          pltpu.SemaphoreType.DMA,
          pltpu.SemaphoreType.DMA,
          pltpu.SemaphoreType.DMA,
          pltpu.SemaphoreType.DMA,
          pltpu.SemaphoreType.DMA,
          pltpu.SemaphoreType.DMA,
          pltpu.SemaphoreType.DMA,
      ],
  )(_sc_body)


_sc_kernel = _make_sc_kernel(P)


def kernel(u, nn_indices, dist_intp_coord_axis1, dist_intp_coord_axis2,
           weight_D, weight_c):
  uT = u.T.astype(jnp.float32)
  idx2 = nn_indices.astype(jnp.int32).reshape(P * NN // 128, 128)
  d1f = dist_intp_coord_axis1[:, 3:6, :].reshape(P * 3 * NN)
  d2f = dist_intp_coord_axis2[:, 3:6, :].reshape(P * 3 * NN)
  D = weight_D.reshape(P).astype(jnp.float32)
  c = weight_c.reshape(P).astype(jnp.float32)
  outT = _sc_kernel(uT, idx2, d1f, d2f, D, c)
  return outT.T


# speedup vs baseline: 1.3394x; 1.3394x over previous
import functools

import jax
import jax.numpy as jnp
from jax import lax
from jax.experimental import pallas as pl
from jax.experimental.pallas import tpu as pltpu
from jax.experimental.pallas import tpu_sc as plsc

P = 100000
NN = 8
T = 16
L = 16
C = 80
NW = 32
GP = 17

H = 0.01
EPS = 1e-8
INV_H_HALF = 0.5 / H
INV_H2 = 1.0 / (H * H)


def _make_sc_kernel(num_points, interpret=False):
  groups = C // L
  nchunks = num_points // C
  base_chunks = nchunks // NW
  extra = nchunks % NW
  dlen = C * 3 * NN

  def _sc_body(uT, idx2, d1f, d2f, Dh, ch, outT,
               idx_v, g_v, g_p, dD_v, d1_v, d2_v, c_v, Dow_v, out_p, out_v,
               isem0, isem1, dsem0, dsem1, gsem0, gsem1, osem0, osem1):
    isem = (isem0, isem1)
    dsem = (dsem0, dsem1)
    gsem = (gsem0, gsem1)
    osem = (osem0, osem1)
    wid = lax.axis_index("s") * 2 + lax.axis_index("c")
    nch = base_chunks + (wid < extra).astype(jnp.int32)
    iota = lax.iota(jnp.int32, L)

    def issue_inputs(chunk, b):
      base = chunk * C
      pltpu.async_copy(
          idx2.at[pl.ds(chunk * groups, groups)], idx_v.at[b], isem[b])
      pltpu.async_copy(d1f.at[pl.ds(base * 3 * NN, dlen)], d1_v.at[b],
                       dsem[b])
      pltpu.async_copy(d2f.at[pl.ds(base * 3 * NN, dlen)], d2_v.at[b],
                       dsem[b])
      pltpu.async_copy(ch.at[pl.ds(base, C)], c_v.at[b], dsem[b])
      pltpu.async_copy(Dh.at[pl.ds(base, C)], Dow_v.at[b], dsem[b])

    def wait_idx(b):
      pltpu.make_async_copy(
          idx2.at[pl.ds(0, groups)], idx_v.at[b], isem[b]).wait()

    def issue_gather(chunk, b):
      base = chunk * C
      for g in range(groups):
        pltpu.async_copy(
            uT.at[idx_v.at[b, g]], g_v.at[b, g, pl.ds(0, NN * L)], gsem[b])
        pltpu.async_copy(
            uT.at[pl.ds(base + g * L, L)],
            g_v.at[b, g, pl.ds(NN * L, L)], gsem[b])
        pltpu.async_copy(Dh.at[idx_v.at[b, g]], dD_v.at[b, g], gsem[b])

    def wait_gather(b):
      for g in range(groups):
        pltpu.make_async_copy(
            uT.at[pl.ds(0, NN * L)],
            g_v.at[b, g, pl.ds(0, NN * L)], gsem[b]).wait()
        pltpu.make_async_copy(
            uT.at[pl.ds(0, L)],
            g_v.at[b, g, pl.ds(NN * L, L)], gsem[b]).wait()
        pltpu.make_async_copy(
            Dh.at[pl.ds(0, NN * L)], dD_v.at[b, g], gsem[b]).wait()

    def wait_inputs(b):
      pltpu.make_async_copy(d1f.at[pl.ds(0, dlen)], d1_v.at[b],
                            dsem[b]).wait()
      pltpu.make_async_copy(d2f.at[pl.ds(0, dlen)], d2_v.at[b],
                            dsem[b]).wait()
      pltpu.make_async_copy(ch.at[pl.ds(0, C)], c_v.at[b], dsem[b]).wait()
      pltpu.make_async_copy(Dh.at[pl.ds(0, C)], Dow_v.at[b], dsem[b]).wait()

    def issue_out(chunk, b):
      pltpu.async_copy(out_v.at[b], outT.at[pl.ds(chunk * C, C)], osem[b])

    def wait_out(b):
      pltpu.make_async_copy(
          out_v.at[b], outT.at[pl.ds(0, C)], osem[b]).wait()

    def compute(chunk, b):
      for g in range(groups):
        plane = g * L + iota
        goff = g * 3 * NN * L

        def axis_weights(dref):
          ws = []
          for i in range(3):
            r = [1.0 / (dref[pl.ds(goff + (i * NN + n) * L, L)] + EPS)
                 for n in range(NN)]
            s = r[0]
            for n in range(1, NN):
              s = s + r[n]
            inv = 1.0 / s
            ws.append([x * inv for x in r])
          a = [(ws[2][n] - ws[0][n]) * INV_H_HALF for n in range(NN)]
          b_ = [(ws[0][n] - 2.0 * ws[1][n] + ws[2][n]) * INV_H2
                for n in range(NN)]
          return a, b_

        a1, b1 = axis_weights(d1_v.at[b])
        a2, b2 = axis_weights(d2_v.at[b])

        Dg = [dD_v[b, g, pl.ds(n * L, L)] for n in range(NN)]
        dD1 = a1[0] * Dg[0]
        dD2 = a2[0] * Dg[0]
        for n in range(1, NN):
          dD1 = dD1 + a1[n] * Dg[n]
          dD2 = dD2 + a2[n] * Dg[n]
        Down = Dow_v[b, pl.ds(g * L, L)]
        W = [dD1 * a1[n] + dD2 * a2[n] + Down * (b1[n] + b2[n])
             for n in range(NN)]
        cown = c_v[b, pl.ds(g * L, L)]

        for r in range((NN + 1) * L):
          g_p[pl.ds(r * GP, T)] = g_v[b, g, r, :]

        rn = [(iota + n * L) * GP for n in range(NN + 1)]
        o17 = plane * GP
        for tt in range(T):
          acc = cown * plsc.load_gather(g_p, [rn[NN] + tt])
          for n in range(NN):
            gv = plsc.load_gather(g_p, [rn[n] + tt])
            acc = acc + W[n] * gv
          plsc.store_scatter(out_p, [o17 + tt], acc)

      for j in range(C):
        out_v[b, j, :] = out_p[pl.ds(j * GP, T)]

    issue_inputs(wid, 0)
    wait_idx(0)
    issue_gather(wid, 0)

    niter = (nch + 1) // 2

    def loop_body(k, carry):
      c0 = wid + NW * (2 * k)
      c1 = wid + NW * (2 * k + 1)
      c2 = wid + NW * (2 * k + 2)
      g1 = (2 * k + 1) < nch
      g2 = (2 * k + 2) < nch

      pl.when(g1)(lambda: issue_inputs(c1, 1))
      wait_inputs(0)
      wait_gather(0)

      def _start_g1():
        wait_idx(1)
        issue_gather(c1, 1)
      pl.when(g1)(_start_g1)

      pl.when(k > 0)(lambda: wait_out(0))
      compute(c0, 0)
      issue_out(c0, 0)

      def _half1():
        pl.when(g2)(lambda: issue_inputs(c2, 0))
        wait_inputs(1)
        wait_gather(1)
        pl.when(k > 0)(lambda: wait_out(1))
        compute(c1, 1)
        issue_out(c1, 1)

        def _start_g2():
          wait_idx(0)
          issue_gather(c2, 0)
        pl.when(g2)(_start_g2)
      pl.when(g1)(_half1)
      return carry

    lax.fori_loop(0, niter, loop_body, jnp.int32(0))
    wait_out(0)
    wait_out(1)

  return functools.partial(
      pl.kernel,
      out_type=jax.ShapeDtypeStruct((num_points, T), jnp.float32),
      mesh=plsc.VectorSubcoreMesh(core_axis_name="c", subcore_axis_name="s",
                                  num_cores=2, num_subcores=16),
      compiler_params=pltpu.CompilerParams(
          needs_layout_passes=False, use_tc_tiling_on_sc=False),
      interpret=interpret,
      scratch_types=[
          pltpu.VMEM((2, groups, 128), jnp.int32),
          pltpu.VMEM((2, groups, (NN + 1) * L, T), jnp.float32),
          pltpu.VMEM(((NN + 1) * L * GP,), jnp.float32),
          pltpu.VMEM((2, groups, NN * L), jnp.float32),
          pltpu.VMEM((2, dlen), jnp.float32),
          pltpu.VMEM((2, dlen), jnp.float32),
          pltpu.VMEM((2, C), jnp.float32),
          pltpu.VMEM((2, C), jnp.float32),
          pltpu.VMEM((C * GP,), jnp.float32),
          pltpu.VMEM((2, C, T), jnp.float32),
          pltpu.SemaphoreType.DMA,
          pltpu.SemaphoreType.DMA,
          pltpu.SemaphoreType.DMA,
          pltpu.SemaphoreType.DMA,
          pltpu.SemaphoreType.DMA,
          pltpu.SemaphoreType.DMA,
          pltpu.SemaphoreType.DMA,
          pltpu.SemaphoreType.DMA,
      ],
  )(_sc_body)


_sc_kernel = _make_sc_kernel(P)


def kernel(u, nn_indices, dist_intp_coord_axis1, dist_intp_coord_axis2,
           weight_D, weight_c):
  uT = u.T.astype(jnp.float32)
  idx2 = (nn_indices.astype(jnp.int32)
          .reshape(P // L, L, NN).transpose(0, 2, 1).reshape(P // L, NN * L))
  d1f = (dist_intp_coord_axis1[:, 3:6, :]
         .reshape(P // L, L, 3 * NN).transpose(0, 2, 1).reshape(P * 3 * NN))
  d2f = (dist_intp_coord_axis2[:, 3:6, :]
         .reshape(P // L, L, 3 * NN).transpose(0, 2, 1).reshape(P * 3 * NN))
  D = weight_D.reshape(P).astype(jnp.float32)
  c = weight_c.reshape(P).astype(jnp.float32)
  outT = _sc_kernel(uT, idx2, d1f, d2f, D, c)
  return outT.T


# speedup vs baseline: 1.4829x; 1.1071x over previous
import functools

import jax
import jax.numpy as jnp
from jax import lax
from jax.experimental import pallas as pl
from jax.experimental.pallas import tpu as pltpu
from jax.experimental.pallas import tpu_sc as plsc

P = 100000
NN = 8
T = 16
L = 16
C = 80
NW = 32
GP = 17

H = 0.01
EPS = 1e-8
INV_H_HALF = 0.5 / H
INV_H2 = 1.0 / (H * H)


def _make_sc_kernel(num_points, interpret=False):
  groups = C // L
  nchunks = num_points // C
  base_chunks = nchunks // NW
  extra = nchunks % NW
  dlen = C * 3 * NN

  def _sc_body(uT, idx2, d1f, d2f, Dh, ch, outT,
               idx_v, g_v, g_p, dD_v, d1_v, d2_v, c_v, Dow_v, out_v,
               isem0, isem1, dsem0, dsem1, gsem0, gsem1, osem0, osem1):
    isem = (isem0, isem1)
    dsem = (dsem0, dsem1)
    gsem = (gsem0, gsem1)
    osem = (osem0, osem1)
    wid = lax.axis_index("s") * 2 + lax.axis_index("c")
    nch = base_chunks + (wid < extra).astype(jnp.int32)
    iota = lax.iota(jnp.int32, L)

    def issue_inputs(chunk, b):
      base = chunk * C
      pltpu.async_copy(
          idx2.at[pl.ds(chunk * groups, groups)], idx_v.at[b], isem[b])
      pltpu.async_copy(d1f.at[pl.ds(base * 3 * NN, dlen)], d1_v.at[b],
                       dsem[b])
      pltpu.async_copy(d2f.at[pl.ds(base * 3 * NN, dlen)], d2_v.at[b],
                       dsem[b])
      pltpu.async_copy(ch.at[pl.ds(base, C)], c_v.at[b], dsem[b])
      pltpu.async_copy(Dh.at[pl.ds(base, C)], Dow_v.at[b], dsem[b])

    def wait_idx(b):
      pltpu.make_async_copy(
          idx2.at[pl.ds(0, groups)], idx_v.at[b], isem[b]).wait()

    def issue_gather(chunk, b):
      base = chunk * C
      for g in range(groups):
        pltpu.async_copy(
            uT.at[idx_v.at[b, g]], g_v.at[b, g, pl.ds(0, NN * L)], gsem[b])
        pltpu.async_copy(
            uT.at[pl.ds(base + g * L, L)],
            g_v.at[b, g, pl.ds(NN * L, L)], gsem[b])
        pltpu.async_copy(Dh.at[idx_v.at[b, g]], dD_v.at[b, g], gsem[b])

    def wait_gather(b):
      for g in range(groups):
        pltpu.make_async_copy(
            uT.at[pl.ds(0, NN * L)],
            g_v.at[b, g, pl.ds(0, NN * L)], gsem[b]).wait()
        pltpu.make_async_copy(
            uT.at[pl.ds(0, L)],
            g_v.at[b, g, pl.ds(NN * L, L)], gsem[b]).wait()
        pltpu.make_async_copy(
            Dh.at[pl.ds(0, NN * L)], dD_v.at[b, g], gsem[b]).wait()

    def wait_inputs(b):
      pltpu.make_async_copy(d1f.at[pl.ds(0, dlen)], d1_v.at[b],
                            dsem[b]).wait()
      pltpu.make_async_copy(d2f.at[pl.ds(0, dlen)], d2_v.at[b],
                            dsem[b]).wait()
      pltpu.make_async_copy(ch.at[pl.ds(0, C)], c_v.at[b], dsem[b]).wait()
      pltpu.make_async_copy(Dh.at[pl.ds(0, C)], Dow_v.at[b], dsem[b]).wait()

    def issue_out(chunk, b):
      for tt in range(T):
        pltpu.async_copy(out_v.at[b, tt],
                         outT.at[tt, pl.ds(chunk * C, C)], osem[b])

    def wait_out(b):
      for tt in range(T):
        pltpu.make_async_copy(out_v.at[b, tt],
                              outT.at[tt, pl.ds(0, C)], osem[b]).wait()

    def compute(chunk, b):
      for g in range(groups):
        plane = g * L + iota
        goff = g * 3 * NN * L

        def axis_weights(dref):
          ws = []
          for i in range(3):
            r = [1.0 / (dref[pl.ds(goff + (i * NN + n) * L, L)] + EPS)
                 for n in range(NN)]
            s = r[0]
            for n in range(1, NN):
              s = s + r[n]
            inv = 1.0 / s
            ws.append([x * inv for x in r])
          a = [(ws[2][n] - ws[0][n]) * INV_H_HALF for n in range(NN)]
          b_ = [(ws[0][n] - 2.0 * ws[1][n] + ws[2][n]) * INV_H2
                for n in range(NN)]
          return a, b_

        a1, b1 = axis_weights(d1_v.at[b])
        a2, b2 = axis_weights(d2_v.at[b])

        Dg = [dD_v[b, g, pl.ds(n * L, L)] for n in range(NN)]
        dD1 = a1[0] * Dg[0]
        dD2 = a2[0] * Dg[0]
        for n in range(1, NN):
          dD1 = dD1 + a1[n] * Dg[n]
          dD2 = dD2 + a2[n] * Dg[n]
        Down = Dow_v[b, pl.ds(g * L, L)]
        W = [dD1 * a1[n] + dD2 * a2[n] + Down * (b1[n] + b2[n])
             for n in range(NN)]
        cown = c_v[b, pl.ds(g * L, L)]

        for r in range((NN + 1) * L):
          g_p[pl.ds(r * GP, T)] = g_v[b, g, r, :]

        rn = [(iota + n * L) * GP for n in range(NN + 1)]
        for tt in range(T):
          acc = cown * plsc.load_gather(g_p, [rn[NN] + tt])
          for n in range(NN):
            gv = plsc.load_gather(g_p, [rn[n] + tt])
            acc = acc + W[n] * gv
          out_v[b, tt, pl.ds(g * L, L)] = acc

    issue_inputs(wid, 0)
    wait_idx(0)
    issue_gather(wid, 0)

    niter = (nch + 1) // 2

    def loop_body(k, carry):
      c0 = wid + NW * (2 * k)
      c1 = wid + NW * (2 * k + 1)
      c2 = wid + NW * (2 * k + 2)
      g1 = (2 * k + 1) < nch
      g2 = (2 * k + 2) < nch

      pl.when(g1)(lambda: issue_inputs(c1, 1))
      wait_inputs(0)
      wait_gather(0)

      def _start_g1():
        wait_idx(1)
        issue_gather(c1, 1)
      pl.when(g1)(_start_g1)

      pl.when(k > 0)(lambda: wait_out(0))
      compute(c0, 0)
      issue_out(c0, 0)

      def _half1():
        pl.when(g2)(lambda: issue_inputs(c2, 0))
        wait_inputs(1)
        wait_gather(1)
        pl.when(k > 0)(lambda: wait_out(1))
        compute(c1, 1)
        issue_out(c1, 1)

        def _start_g2():
          wait_idx(0)
          issue_gather(c2, 0)
        pl.when(g2)(_start_g2)
      pl.when(g1)(_half1)
      return carry

    lax.fori_loop(0, niter, loop_body, jnp.int32(0))
    wait_out(0)
    wait_out(1)

  return functools.partial(
      pl.kernel,
      out_type=jax.ShapeDtypeStruct((T, num_points), jnp.float32),
      mesh=plsc.VectorSubcoreMesh(core_axis_name="c", subcore_axis_name="s",
                                  num_cores=2, num_subcores=16),
      compiler_params=pltpu.CompilerParams(
          needs_layout_passes=False, use_tc_tiling_on_sc=False),
      interpret=interpret,
      scratch_types=[
          pltpu.VMEM((2, groups, 128), jnp.int32),
          pltpu.VMEM((2, groups, (NN + 1) * L, T), jnp.float32),
          pltpu.VMEM(((NN + 1) * L * GP,), jnp.float32),
          pltpu.VMEM((2, groups, NN * L), jnp.float32),
          pltpu.VMEM((2, dlen), jnp.float32),
          pltpu.VMEM((2, dlen), jnp.float32),
          pltpu.VMEM((2, C), jnp.float32),
          pltpu.VMEM((2, C), jnp.float32),
          pltpu.VMEM((2, T, C), jnp.float32),
          pltpu.SemaphoreType.DMA,
          pltpu.SemaphoreType.DMA,
          pltpu.SemaphoreType.DMA,
          pltpu.SemaphoreType.DMA,
          pltpu.SemaphoreType.DMA,
          pltpu.SemaphoreType.DMA,
          pltpu.SemaphoreType.DMA,
          pltpu.SemaphoreType.DMA,
      ],
  )(_sc_body)


_sc_kernel = _make_sc_kernel(P)


def kernel(u, nn_indices, dist_intp_coord_axis1, dist_intp_coord_axis2,
           weight_D, weight_c):
  uT = u.T.astype(jnp.float32)
  idx2 = (nn_indices.astype(jnp.int32)
          .reshape(P // L, L, NN).transpose(0, 2, 1).reshape(P // L, NN * L))
  d1f = (dist_intp_coord_axis1[:, 3:6, :]
         .reshape(P // L, L, 3 * NN).transpose(0, 2, 1).reshape(P * 3 * NN))
  d2f = (dist_intp_coord_axis2[:, 3:6, :]
         .reshape(P // L, L, 3 * NN).transpose(0, 2, 1).reshape(P * 3 * NN))
  D = weight_D.reshape(P).astype(jnp.float32)
  c = weight_c.reshape(P).astype(jnp.float32)
  return _sc_kernel(uT, idx2, d1f, d2f, D, c)
